# baseline (speedup 1.0000x reference)
import functools

import jax
import jax.numpy as jnp
from jax import lax
from jax.experimental import pallas as pl
from jax.experimental.pallas import tpu as pltpu
from jax.experimental.pallas import tpu_sc as plsc

_N = 10001
_NP = 10240
_D = 128
_R = 50
_NC = 2
_NS = 16
_NW = _NC * _NS
_K = 24
_NB = 6
_PD = 4
_NCH = 432
_NG = _NCH // _NB
_E = 320000
_EP = _NW * _NCH * _K
_ROWS_PER_TILE = _NP // _NS
_B = 1024
_QB = (2 * _B) // _NW
_NSCALE = float(1 << 23)

_mesh = plsc.VectorSubcoreMesh(core_axis_name="c", subcore_axis_name="s")


def _compute_msgs(e_v, rows_v, m_v, r_v):
    for base, lo in ((0, 0), (8, 8)):
        tv = e_v[2, pl.ds(base, 16)]
        nv = e_v[3, pl.ds(base, 16)].astype(jnp.float32) * (1.0 / _NSCALE)
        for l in range(lo, 16):
            ns = nv[l]
            te = tv[l]
            e = base + l
            for j in range(8):
                sl = pl.ds(j * 16, 16)
                m_v[e, sl] = (rows_v[e, sl] - r_v[te, sl]) * ns


@functools.partial(
    pl.kernel,
    out_type=jax.ShapeDtypeStruct((_NC, _NP, _D), jnp.float32),
    mesh=_mesh,
    scratch_types=(
        tuple(pltpu.VMEM((4, _K), jnp.int32) for _ in range(_NB)),
        tuple(pltpu.VMEM((_K, _D), jnp.float32) for _ in range(_NB)),
        tuple(pltpu.VMEM((_K, _D), jnp.float32) for _ in range(_NB)),
        pltpu.VMEM((_R, _D), jnp.float32),
        pltpu.VMEM_SHARED((_NP, _D), jnp.float32),
        tuple(pltpu.SemaphoreType.DMA for _ in range(_NB)),
        tuple(pltpu.SemaphoreType.DMA for _ in range(_NB)),
    ),
)
def _sc_edge_pass(x_hbm, r_hbm, eidx_hbm, zrows_hbm,
                  agg_out,
                  ebufs, rbufs, mbufs, r_v, agg_sh, gsems, ssems):
    cid = lax.axis_index("c")
    sid = lax.axis_index("s")
    wid = sid * _NC + cid
    slab = eidx_hbm.at[wid]

    pltpu.sync_copy(r_hbm, r_v)
    pltpu.sync_copy(zrows_hbm, agg_sh.at[pl.ds(sid * _ROWS_PER_TILE, _ROWS_PER_TILE)])
    plsc.subcore_barrier()

    def _prefetch(ci, b):
        pltpu.sync_copy(slab.at[ci], ebufs[b])
        pltpu.async_copy(x_hbm.at[ebufs[b].at[0]], rbufs[b], gsems[b])

    def _drain_scatter(b):
        pltpu.make_async_copy(mbufs[b], agg_sh.at[ebufs[b].at[1]], ssems[b]).wait()

    for b in range(_PD):
        _prefetch(b, b)

    def _grp(p, carry):
        for b in range(_NB):
            c = _NB * p + b
            bn = (b + _PD) % _NB
            pltpu.make_async_copy(x_hbm.at[ebufs[b].at[0]], rbufs[b], gsems[b]).wait()

            if b < _NB - _PD:
                @pl.when(p > 0)
                def _(bn=bn):
                    _drain_scatter(bn)

                _prefetch(c + _PD, bn)
            else:
                @pl.when(p < _NG - 1)
                def _(c=c, bn=bn):
                    _drain_scatter(bn)
                    _prefetch(c + _PD, bn)

            _compute_msgs(ebufs[b], rbufs[b], mbufs[b], r_v)
            pltpu.async_copy(mbufs[b], agg_sh.at[ebufs[b].at[1]], ssems[b], add=True)
        return carry

    lax.fori_loop(0, _NG, _grp, 0)

    for b in range(_NB):
        _drain_scatter(b)
    plsc.subcore_barrier()

    pltpu.sync_copy(agg_sh.at[pl.ds(sid * _ROWS_PER_TILE, _ROWS_PER_TILE)],
                    agg_out.at[cid].at[pl.ds(sid * _ROWS_PER_TILE, _ROWS_PER_TILE)])


@functools.partial(
    pl.kernel,
    out_type=jax.ShapeDtypeStruct((2 * _B, _D), jnp.float32),
    mesh=_mesh,
    scratch_types=(
        pltpu.VMEM((_QB,), jnp.int32),
        pltpu.VMEM((_QB, _D), jnp.float32),
        pltpu.SemaphoreType.DMA,
    ),
)
def _sc_rowgather(x_hbm, q_hbm, out_hbm, qv, rowsv, sem):
    cid = lax.axis_index("c")
    sid = lax.axis_index("s")
    wid = sid * _NC + cid
    base = wid * _QB
    pltpu.sync_copy(q_hbm.at[pl.ds(base, _QB)], qv)
    pltpu.async_copy(x_hbm.at[qv], rowsv, sem).wait()
    pltpu.sync_copy(rowsv, out_hbm.at[pl.ds(base, _QB)])


_BR = 256


def _tc_combine_body(a_ref, x_ref, r_ref, w_ref, wr_ref, xo_ref, ro_ref):
    u = a_ref[0] + a_ref[1] + x_ref[...]
    xo_ref[...] = jnp.tanh(jnp.dot(u, w_ref[...], preferred_element_type=jnp.float32))
    ro_ref[...] = jnp.dot(r_ref[...], wr_ref[...], preferred_element_type=jnp.float32)


def _tc_combine(agg, x, r, w, wr):
    return pl.pallas_call(
        _tc_combine_body,
        grid=(_NP // _BR,),
        in_specs=[
            pl.BlockSpec((_NC, _BR, _D), lambda i: (0, i, 0)),
            pl.BlockSpec((_BR, _D), lambda i: (i, 0)),
            pl.BlockSpec((_R, _D), lambda i: (0, 0)),
            pl.BlockSpec((_D, _D), lambda i: (0, 0)),
            pl.BlockSpec((_D, _D), lambda i: (0, 0)),
        ],
        out_specs=[
            pl.BlockSpec((_BR, _D), lambda i: (i, 0)),
            pl.BlockSpec((_R, _D), lambda i: (0, 0)),
        ],
        out_shape=[
            jax.ShapeDtypeStruct((_NP, _D), jnp.float32),
            jax.ShapeDtypeStruct((_R, _D), jnp.float32),
        ],
    )(agg, x, r, w, wr)


def kernel(init_embed, init_rel, W0, Wr0, W1, Wr1, edge_norm, edge_index, edge_type, subj, obj):
    x0 = jnp.pad(init_embed.astype(jnp.float32), ((0, _NP - _N), (0, 0)))
    src = edge_index[0].astype(jnp.int32)
    dst = edge_index[1].astype(jnp.int32)
    et = edge_type.astype(jnp.int32)
    nrm_q = jnp.round(edge_norm.astype(jnp.float32) * _NSCALE).astype(jnp.int32)
    pad = _EP - _E
    src_p = jnp.pad(src, (0, pad)).reshape(_NW, _NCH, _K)
    dst_p = jnp.pad(dst, (0, pad)).reshape(_NW, _NCH, _K)
    et_p = jnp.pad(et, (0, pad)).reshape(_NW, _NCH, _K)
    nrm_p = jnp.pad(nrm_q, (0, pad)).reshape(_NW, _NCH, _K)
    eidx = jnp.stack([src_p, dst_p, et_p, nrm_p], axis=2)
    zrows = jnp.zeros((_ROWS_PER_TILE, _D), jnp.float32)

    r0 = init_rel.astype(jnp.float32)
    agg = _sc_edge_pass(x0, r0, eidx, zrows)
    x1, r1 = _tc_combine(agg, x0, r0, W0, Wr0)
    agg = _sc_edge_pass(x1, r1, eidx, zrows)
    x2, r2 = _tc_combine(agg, x1, r1, W1, Wr1)

    q = jnp.concatenate([subj.astype(jnp.int32), obj.astype(jnp.int32)])
    qe = _sc_rowgather(x2, q)
    return (qe[:_B], qe[_B:], x2[:_N], r2)


# speedup vs baseline: 1.4246x; 1.4246x over previous
import functools

import jax
import jax.numpy as jnp
from jax import lax
from jax.experimental import pallas as pl
from jax.experimental.pallas import tpu as pltpu
from jax.experimental.pallas import tpu_sc as plsc

_N = 10001
_NP = 10240
_D = 128
_R = 50
_NC = 2
_NS = 16
_NW = _NC * _NS
_K = 48
_NB = 6
_PD = 4
_NCH = 216
_NG = _NCH // _NB
_E = 320000
_EP = _NW * _NCH * _K
_ROWS_PER_TILE = _NP // _NS
_B = 1024
_QB = (2 * _B) // _NW
_NSCALE = float(1 << 23)

_mesh = plsc.VectorSubcoreMesh(core_axis_name="c", subcore_axis_name="s")


def _compute_msgs(e_v, rows_v, r_v):
    def _grp16(g, carry):
        tv = e_v[2, pl.ds(g * 16, 16)]
        nv = e_v[3, pl.ds(g * 16, 16)].astype(jnp.float32) * (1.0 / _NSCALE)
        for l in range(16):
            ns = nv[l]
            e = g * 16 + l
            row = rows_v.at[e]
            rrow = r_v.at[tv[l]]
            for j in range(8):
                sl = pl.ds(j * 16, 16)
                row[sl] = (row[sl] - rrow[sl]) * ns
        return carry

    lax.fori_loop(0, _K // 16, _grp16, 0)


@functools.partial(
    pl.kernel,
    out_type=jax.ShapeDtypeStruct((_NC, _NP, _D), jnp.float32),
    mesh=_mesh,
    scratch_types=(
        tuple(pltpu.VMEM((4, _K), jnp.int32) for _ in range(_NB)),
        tuple(pltpu.VMEM((_K, _D), jnp.float32) for _ in range(_NB)),
        pltpu.VMEM((_R, _D), jnp.float32),
        pltpu.VMEM_SHARED((_NP, _D), jnp.float32),
        tuple(pltpu.SemaphoreType.DMA for _ in range(_NB)),
        tuple(pltpu.SemaphoreType.DMA for _ in range(_NB)),
    ),
)
def _sc_edge_pass(x_hbm, r_hbm, eidx_hbm, zrows_hbm,
                  agg_out,
                  ebufs, rbufs, r_v, agg_sh, gsems, ssems):
    cid = lax.axis_index("c")
    sid = lax.axis_index("s")
    wid = sid * _NC + cid
    slab = eidx_hbm.at[wid]

    pltpu.sync_copy(r_hbm, r_v)
    pltpu.sync_copy(zrows_hbm, agg_sh.at[pl.ds(sid * _ROWS_PER_TILE, _ROWS_PER_TILE)])
    plsc.subcore_barrier()

    def _prefetch(ci, b):
        pltpu.sync_copy(slab.at[ci], ebufs[b])
        pltpu.async_copy(x_hbm.at[ebufs[b].at[0]], rbufs[b], gsems[b])

    def _drain_scatter(b):
        pltpu.make_async_copy(rbufs[b], agg_sh.at[ebufs[b].at[1]], ssems[b]).wait()

    for b in range(_PD):
        _prefetch(b, b)

    def _grp(p, carry):
        for b in range(_NB):
            c = _NB * p + b
            bn = (b + _PD) % _NB
            pltpu.make_async_copy(x_hbm.at[ebufs[b].at[0]], rbufs[b], gsems[b]).wait()

            if b < _NB - _PD:
                @pl.when(p > 0)
                def _(bn=bn):
                    _drain_scatter(bn)

                _prefetch(c + _PD, bn)
            else:
                @pl.when(p < _NG - 1)
                def _(c=c, bn=bn):
                    _drain_scatter(bn)
                    _prefetch(c + _PD, bn)

            _compute_msgs(ebufs[b], rbufs[b], r_v)
            pltpu.async_copy(rbufs[b], agg_sh.at[ebufs[b].at[1]], ssems[b], add=True)
        return carry

    lax.fori_loop(0, _NG, _grp, 0)

    for b in range(_NB):
        _drain_scatter(b)
    plsc.subcore_barrier()

    pltpu.sync_copy(agg_sh.at[pl.ds(sid * _ROWS_PER_TILE, _ROWS_PER_TILE)],
                    agg_out.at[cid].at[pl.ds(sid * _ROWS_PER_TILE, _ROWS_PER_TILE)])


@functools.partial(
    pl.kernel,
    out_type=jax.ShapeDtypeStruct((2 * _B, _D), jnp.float32),
    mesh=_mesh,
    scratch_types=(
        pltpu.VMEM((_QB,), jnp.int32),
        pltpu.VMEM((_QB, _D), jnp.float32),
        pltpu.SemaphoreType.DMA,
    ),
)
def _sc_rowgather(x_hbm, q_hbm, out_hbm, qv, rowsv, sem):
    cid = lax.axis_index("c")
    sid = lax.axis_index("s")
    wid = sid * _NC + cid
    base = wid * _QB
    pltpu.sync_copy(q_hbm.at[pl.ds(base, _QB)], qv)
    pltpu.async_copy(x_hbm.at[qv], rowsv, sem).wait()
    pltpu.sync_copy(rowsv, out_hbm.at[pl.ds(base, _QB)])


_BR = 256


def _tc_combine_body(a_ref, x_ref, r_ref, w_ref, wr_ref, xo_ref, ro_ref):
    u = a_ref[0] + a_ref[1] + x_ref[...]
    xo_ref[...] = jnp.tanh(jnp.dot(u, w_ref[...], preferred_element_type=jnp.float32))
    ro_ref[...] = jnp.dot(r_ref[...], wr_ref[...], preferred_element_type=jnp.float32)


def _tc_combine(agg, x, r, w, wr):
    return pl.pallas_call(
        _tc_combine_body,
        grid=(_NP // _BR,),
        in_specs=[
            pl.BlockSpec((_NC, _BR, _D), lambda i: (0, i, 0)),
            pl.BlockSpec((_BR, _D), lambda i: (i, 0)),
            pl.BlockSpec((_R, _D), lambda i: (0, 0)),
            pl.BlockSpec((_D, _D), lambda i: (0, 0)),
            pl.BlockSpec((_D, _D), lambda i: (0, 0)),
        ],
        out_specs=[
            pl.BlockSpec((_BR, _D), lambda i: (i, 0)),
            pl.BlockSpec((_R, _D), lambda i: (0, 0)),
        ],
        out_shape=[
            jax.ShapeDtypeStruct((_NP, _D), jnp.float32),
            jax.ShapeDtypeStruct((_R, _D), jnp.float32),
        ],
    )(agg, x, r, w, wr)


def kernel(init_embed, init_rel, W0, Wr0, W1, Wr1, edge_norm, edge_index, edge_type, subj, obj):
    x0 = jnp.pad(init_embed.astype(jnp.float32), ((0, _NP - _N), (0, 0)))
    src = edge_index[0].astype(jnp.int32)
    dst = edge_index[1].astype(jnp.int32)
    et = edge_type.astype(jnp.int32)
    nrm_q = jnp.round(edge_norm.astype(jnp.float32) * _NSCALE).astype(jnp.int32)
    pad = _EP - _E
    src_p = jnp.pad(src, (0, pad)).reshape(_NW, _NCH, _K)
    dst_p = jnp.pad(dst, (0, pad)).reshape(_NW, _NCH, _K)
    et_p = jnp.pad(et, (0, pad)).reshape(_NW, _NCH, _K)
    nrm_p = jnp.pad(nrm_q, (0, pad)).reshape(_NW, _NCH, _K)
    eidx = jnp.stack([src_p, dst_p, et_p, nrm_p], axis=2)
    zrows = jnp.zeros((_ROWS_PER_TILE, _D), jnp.float32)

    r0 = init_rel.astype(jnp.float32)
    agg = _sc_edge_pass(x0, r0, eidx, zrows)
    x1, r1 = _tc_combine(agg, x0, r0, W0, Wr0)
    agg = _sc_edge_pass(x1, r1, eidx, zrows)
    x2, r2 = _tc_combine(agg, x1, r1, W1, Wr1)

    q = jnp.concatenate([subj.astype(jnp.int32), obj.astype(jnp.int32)])
    qe = _sc_rowgather(x2, q)
    return (qe[:_B], qe[_B:], x2[:_N], r2)


# speedup vs baseline: 1.4729x; 1.0339x over previous
import functools

import jax
import jax.numpy as jnp
from jax import lax
from jax.experimental import pallas as pl
from jax.experimental.pallas import tpu as pltpu
from jax.experimental.pallas import tpu_sc as plsc

_N = 10001
_NP = 10240
_D = 128
_R = 50
_NC = 2
_NS = 16
_NW = _NC * _NS
_K = 96
_NB = 3
_NCH = 108
_NG = _NCH // _NB
_E = 320000
_EP = _NW * _NCH * _K
_ROWS_PER_TILE = _NP // _NS
_B = 1024
_QB = (2 * _B) // _NW

_mesh = plsc.VectorSubcoreMesh(core_axis_name="c", subcore_axis_name="s")


def _compute_msgs(e_v, n_v, rows_v, r_v):
    def _msg(g, c2):
        tv = e_v[2, pl.ds(g * 16, 16)]
        nv = n_v[pl.ds(g * 16, 16)]
        for l in range(16):
            ns = nv[l]
            te = tv[l]
            e = g * 16 + l
            for j in range(8):
                sl = pl.ds(j * 16, 16)
                rows_v[e, sl] = (rows_v[e, sl] - r_v[te, sl]) * ns
        return c2

    lax.fori_loop(0, _K // 16, _msg, 0)


@functools.partial(
    pl.kernel,
    out_type=jax.ShapeDtypeStruct((_NC, _NP, _D), jnp.float32),
    mesh=_mesh,
    scratch_types=(
        pltpu.VMEM((4, _K), jnp.int32),
        pltpu.VMEM((4, _K), jnp.int32),
        pltpu.VMEM((4, _K), jnp.int32),
        pltpu.VMEM((_K,), jnp.float32),
        pltpu.VMEM((_K,), jnp.float32),
        pltpu.VMEM((_K,), jnp.float32),
        pltpu.VMEM((_K, _D), jnp.float32),
        pltpu.VMEM((_K, _D), jnp.float32),
        pltpu.VMEM((_K, _D), jnp.float32),
        pltpu.VMEM((_R, _D), jnp.float32),
        pltpu.VMEM_SHARED((_NP, _D), jnp.float32),
        pltpu.SemaphoreType.DMA,
        pltpu.SemaphoreType.DMA,
        pltpu.SemaphoreType.DMA,
        pltpu.SemaphoreType.DMA,
        pltpu.SemaphoreType.DMA,
        pltpu.SemaphoreType.DMA,
    ),
)
def _sc_edge_pass(x_hbm, r_hbm, eidx_hbm, nrm_hbm, zrows_hbm,
                  agg_out,
                  e0, e1, e2, n0, n1, n2, rows0, rows1, rows2, r_v, agg_sh,
                  gs0, gs1, gs2, ss0, ss1, ss2):
    cid = lax.axis_index("c")
    sid = lax.axis_index("s")
    wid = sid * _NC + cid
    slab = eidx_hbm.at[wid]
    nslab = nrm_hbm.at[wid]
    ebufs = (e0, e1, e2)
    nbufs = (n0, n1, n2)
    rbufs = (rows0, rows1, rows2)
    gsems = (gs0, gs1, gs2)
    ssems = (ss0, ss1, ss2)

    pltpu.sync_copy(r_hbm, r_v)
    pltpu.sync_copy(zrows_hbm, agg_sh.at[pl.ds(sid * _ROWS_PER_TILE, _ROWS_PER_TILE)])
    plsc.subcore_barrier()

    def _prefetch(ci, b):
        pltpu.sync_copy(slab.at[ci], ebufs[b])
        pltpu.sync_copy(nslab.at[ci], nbufs[b])
        pltpu.async_copy(x_hbm.at[ebufs[b].at[0]], rbufs[b], gsems[b])

    def _drain_scatter(b):
        pltpu.make_async_copy(rbufs[b], agg_sh.at[ebufs[b].at[1]], ssems[b]).wait()

    _prefetch(0, 0)
    _prefetch(1, 1)

    def _grp(p, carry):
        for b in range(_NB):
            c = _NB * p + b
            pltpu.make_async_copy(x_hbm.at[ebufs[b].at[0]], rbufs[b], gsems[b]).wait()
            _compute_msgs(ebufs[b], nbufs[b], rbufs[b], r_v)
            pltpu.async_copy(rbufs[b], agg_sh.at[ebufs[b].at[1]], ssems[b], add=True)

            bp = (b + _NB - 1) % _NB
            if b == 0:
                @pl.when(p > 0)
                def _(bp=bp):
                    _drain_scatter(bp)

                _prefetch(c + 2, bp)
            else:
                @pl.when(p < _NG - 1)
                def _(c=c, bp=bp):
                    _drain_scatter(bp)
                    _prefetch(c + 2, bp)
        return carry

    lax.fori_loop(0, _NG, _grp, 0)

    for b in range(_NB):
        _drain_scatter(b)
    plsc.subcore_barrier()

    pltpu.sync_copy(agg_sh.at[pl.ds(sid * _ROWS_PER_TILE, _ROWS_PER_TILE)],
                    agg_out.at[cid].at[pl.ds(sid * _ROWS_PER_TILE, _ROWS_PER_TILE)])


@functools.partial(
    pl.kernel,
    out_type=jax.ShapeDtypeStruct((2 * _B, _D), jnp.float32),
    mesh=_mesh,
    scratch_types=(
        pltpu.VMEM((_QB,), jnp.int32),
        pltpu.VMEM((_QB, _D), jnp.float32),
        pltpu.SemaphoreType.DMA,
    ),
)
def _sc_rowgather(x_hbm, q_hbm, out_hbm, qv, rowsv, sem):
    cid = lax.axis_index("c")
    sid = lax.axis_index("s")
    wid = sid * _NC + cid
    base = wid * _QB
    pltpu.sync_copy(q_hbm.at[pl.ds(base, _QB)], qv)
    pltpu.async_copy(x_hbm.at[qv], rowsv, sem).wait()
    pltpu.sync_copy(rowsv, out_hbm.at[pl.ds(base, _QB)])


_BR = 256


def _tc_combine_body(a_ref, x_ref, r_ref, w_ref, wr_ref, xo_ref, ro_ref):
    u = a_ref[0] + a_ref[1] + x_ref[...]
    xo_ref[...] = jnp.tanh(jnp.dot(u, w_ref[...], preferred_element_type=jnp.float32))
    ro_ref[...] = jnp.dot(r_ref[...], wr_ref[...], preferred_element_type=jnp.float32)


def _tc_combine(agg, x, r, w, wr):
    return pl.pallas_call(
        _tc_combine_body,
        grid=(_NP // _BR,),
        in_specs=[
            pl.BlockSpec((_NC, _BR, _D), lambda i: (0, i, 0)),
            pl.BlockSpec((_BR, _D), lambda i: (i, 0)),
            pl.BlockSpec((_R, _D), lambda i: (0, 0)),
            pl.BlockSpec((_D, _D), lambda i: (0, 0)),
            pl.BlockSpec((_D, _D), lambda i: (0, 0)),
        ],
        out_specs=[
            pl.BlockSpec((_BR, _D), lambda i: (i, 0)),
            pl.BlockSpec((_R, _D), lambda i: (0, 0)),
        ],
        out_shape=[
            jax.ShapeDtypeStruct((_NP, _D), jnp.float32),
            jax.ShapeDtypeStruct((_R, _D), jnp.float32),
        ],
    )(agg, x, r, w, wr)


def kernel(init_embed, init_rel, W0, Wr0, W1, Wr1, edge_norm, edge_index, edge_type, subj, obj):
    x0 = jnp.pad(init_embed.astype(jnp.float32), ((0, _NP - _N), (0, 0)))
    src = edge_index[0].astype(jnp.int32)
    dst = edge_index[1].astype(jnp.int32)
    et = edge_type.astype(jnp.int32)
    nrm = edge_norm.astype(jnp.float32)
    pad = _EP - _E
    src_p = jnp.pad(src, (0, pad)).reshape(_NW, _NCH, _K)
    dst_p = jnp.pad(dst, (0, pad)).reshape(_NW, _NCH, _K)
    et_p = jnp.pad(et, (0, pad)).reshape(_NW, _NCH, _K)
    nrm_p = jnp.pad(nrm, (0, pad)).reshape(_NW, _NCH, _K)
    eidx = jnp.stack([src_p, dst_p, et_p, et_p], axis=2)
    zrows = jnp.zeros((_ROWS_PER_TILE, _D), jnp.float32)

    r0 = init_rel.astype(jnp.float32)
    agg = _sc_edge_pass(x0, r0, eidx, nrm_p, zrows)
    x1, r1 = _tc_combine(agg, x0, r0, W0, Wr0)
    agg = _sc_edge_pass(x1, r1, eidx, nrm_p, zrows)
    x2, r2 = _tc_combine(agg, x1, r1, W1, Wr1)

    q = jnp.concatenate([subj.astype(jnp.int32), obj.astype(jnp.int32)])
    qe = _sc_rowgather(x2, q)
    return (qe[:_B], qe[_B:], x2[:_N], r2)
